# baseline (speedup 1.0000x reference)
import functools

import jax
import jax.numpy as jnp
from jax import lax
from jax.experimental import pallas as pl
from jax.experimental.pallas import tpu as pltpu

EPS_ = 1e-10


def _loss_kernel(loc_ref, *refs, tile_r, hw, nb, nt):
    rm_refs = refs[:nb]
    b_refs = refs[nb:2 * nb]
    out_ref = refs[2 * nb]
    num_acc, den_acc = refs[2 * nb + 1], refs[2 * nb + 2]
    t = pl.program_id(0)

    @pl.when(t == 0)
    def _init():
        num_acc[0] = 0.0
        den_acc[0] = 0.0

    col = lax.broadcasted_iota(jnp.int32, (tile_r, hw), 1)

    num = 0.0
    den = 0.0
    for k in range(nb):
        x = rm_refs[k][...]
        loc = loc_ref[0, 0, k * tile_r:(k + 1) * tile_r]
        bmap = b_refs[k][0]
        mask = col == loc[:, None]

        denom = jnp.sum(x, axis=1)
        srm = jnp.sum(jnp.where(mask, x, 0.0), axis=1)
        sb = jnp.sum(jnp.where(mask, bmap, 0.0), axis=1)

        num += jnp.sum(sb * -jnp.log(EPS_ + srm / denom))
        den += jnp.sum(sb)

    num_acc[0] += num
    den_acc[0] += den

    @pl.when(t == nt - 1)
    def _fin():
        out_ref[...] = jnp.full((1, 1), num_acc[0] / (1.0 + den_acc[0]), jnp.float32)


def kernel(response_map, source_feature_1d_locations, boundaries):
    B, S, H, W = response_map.shape
    HW = H * W
    TILE_R = 32
    T = S // TILE_R

    rm = response_map.reshape(B * S, HW)
    bnd = boundaries.reshape(B, 1, HW)
    loc = (source_feature_1d_locations.astype(jnp.int32)
           .reshape(B, T, TILE_R).transpose(1, 0, 2).reshape(T, 1, B * TILE_R))

    rm_specs = [
        pl.BlockSpec((TILE_R, HW), functools.partial(lambda k, t: (k * T + t, 0), k))
        for k in range(B)
    ]
    b_specs = [
        pl.BlockSpec((1, 1, HW), functools.partial(lambda k, t: (k, 0, 0), k))
        for k in range(B)
    ]

    out = pl.pallas_call(
        functools.partial(_loss_kernel, tile_r=TILE_R, hw=HW, nb=B, nt=T),
        grid=(T,),
        in_specs=[pl.BlockSpec((1, 1, B * TILE_R), lambda t: (t, 0, 0))] + rm_specs + b_specs,
        out_specs=pl.BlockSpec((1, 1), lambda t: (0, 0)),
        out_shape=jax.ShapeDtypeStruct((1, 1), jnp.float32),
        scratch_shapes=[
            pltpu.SMEM((1,), jnp.float32),
            pltpu.SMEM((1,), jnp.float32),
        ],
    )(loc, *([rm] * B), *([bnd] * B))
    return out[0, 0]


# speedup vs baseline: 1.4592x; 1.4592x over previous
import functools

import jax
import jax.numpy as jnp
from jax import lax
from jax.experimental import pallas as pl
from jax.experimental.pallas import tpu as pltpu

EPS_ = 1e-10


def _loss_kernel(row_ref, col_ref, rm_ref, b_ref, out_ref, num_acc, den_acc,
                 *, tile_r, h, w, nb, nt):
    b = pl.program_id(0)
    t = pl.program_id(1)

    @pl.when(jnp.logical_and(b == 0, t == 0))
    def _init():
        num_acc[0] = 0.0
        den_acc[0] = 0.0

    x = rm_ref[0]
    bmap = b_ref[0, 0]
    row = row_ref[0, 0]
    col = col_ref[0, 0]

    iota_w = lax.broadcasted_iota(jnp.int32, (tile_r, 1, w), 2)
    mask_w = iota_w == col[:, None, None]
    iota_h = lax.broadcasted_iota(jnp.int32, (tile_r, h), 1)
    mask_h = iota_h == row[:, None]

    sum_w = jnp.sum(x, axis=2)
    denom = jnp.sum(sum_w, axis=1)

    srm_w = jnp.sum(jnp.where(mask_w, x, 0.0), axis=2)
    srm = jnp.sum(jnp.where(mask_h, srm_w, 0.0), axis=1)

    sb_w = jnp.sum(jnp.where(mask_w, bmap[None], 0.0), axis=2)
    sb = jnp.sum(jnp.where(mask_h, sb_w, 0.0), axis=1)

    num_acc[0] += jnp.sum(sb * -jnp.log(EPS_ + srm / denom))
    den_acc[0] += jnp.sum(sb)

    @pl.when(jnp.logical_and(b == nb - 1, t == nt - 1))
    def _fin():
        out_ref[...] = jnp.full((1, 1), num_acc[0] / (1.0 + den_acc[0]), jnp.float32)


def kernel(response_map, source_feature_1d_locations, boundaries):
    B, S, H, W = response_map.shape
    TILE_R = 32
    T = S // TILE_R

    loc = source_feature_1d_locations.astype(jnp.int32)
    row = (loc // W).reshape(B * T, 1, TILE_R)
    col = (loc % W).reshape(B * T, 1, TILE_R)

    out = pl.pallas_call(
        functools.partial(_loss_kernel, tile_r=TILE_R, h=H, w=W, nb=B, nt=T),
        grid=(B, T),
        in_specs=[
            pl.BlockSpec((1, 1, TILE_R), lambda b, t: (b * T + t, 0, 0)),
            pl.BlockSpec((1, 1, TILE_R), lambda b, t: (b * T + t, 0, 0)),
            pl.BlockSpec((1, TILE_R, H, W), lambda b, t: (b, t, 0, 0)),
            pl.BlockSpec((1, 1, H, W), lambda b, t: (b, 0, 0, 0)),
        ],
        out_specs=pl.BlockSpec((1, 1), lambda b, t: (0, 0)),
        out_shape=jax.ShapeDtypeStruct((1, 1), jnp.float32),
        scratch_shapes=[
            pltpu.SMEM((1,), jnp.float32),
            pltpu.SMEM((1,), jnp.float32),
        ],
    )(row, col, response_map, boundaries)
    return out[0, 0]


# speedup vs baseline: 1.5268x; 1.0463x over previous
import functools

import jax
import jax.numpy as jnp
from jax import lax
from jax.experimental import pallas as pl
from jax.experimental.pallas import tpu as pltpu

EPS_ = 1e-10


def _loss_kernel(row_ref, col_ref, rm_ref, b_ref, out_ref, num_acc, den_acc,
                 *, tile_r, h, w, nb, nt):
    b = pl.program_id(0)
    t = pl.program_id(1)

    @pl.when(jnp.logical_and(b == 0, t == 0))
    def _init():
        num_acc[0] = 0.0
        den_acc[0] = 0.0

    x = rm_ref[0]
    bmap = b_ref[0, 0]
    row = row_ref[0, 0]
    col = col_ref[0, 0]

    iota_w = lax.broadcasted_iota(jnp.int32, (tile_r, 1, w), 2)
    mask_w = iota_w == col[:, None, None]
    iota_h = lax.broadcasted_iota(jnp.int32, (tile_r, h), 1)
    mask_h = iota_h == row[:, None]

    sum_w = jnp.sum(x, axis=2)
    denom = jnp.sum(sum_w, axis=1)

    srm_w = jnp.sum(jnp.where(mask_w, x, 0.0), axis=2)
    srm = jnp.sum(jnp.where(mask_h, srm_w, 0.0), axis=1)

    sb_w = jnp.sum(jnp.where(mask_w, bmap[None], 0.0), axis=2)
    sb = jnp.sum(jnp.where(mask_h, sb_w, 0.0), axis=1)

    num_acc[0] += jnp.sum(sb * -jnp.log(EPS_ + srm / denom))
    den_acc[0] += jnp.sum(sb)

    @pl.when(jnp.logical_and(b == nb - 1, t == nt - 1))
    def _fin():
        out_ref[...] = jnp.full((1, 1), num_acc[0] / (1.0 + den_acc[0]), jnp.float32)


def kernel(response_map, source_feature_1d_locations, boundaries):
    B, S, H, W = response_map.shape
    TILE_R = 64
    T = S // TILE_R

    loc = source_feature_1d_locations.astype(jnp.int32)
    row = (loc // W).reshape(B * T, 1, TILE_R)
    col = (loc % W).reshape(B * T, 1, TILE_R)

    out = pl.pallas_call(
        functools.partial(_loss_kernel, tile_r=TILE_R, h=H, w=W, nb=B, nt=T),
        grid=(B, T),
        in_specs=[
            pl.BlockSpec((1, 1, TILE_R), lambda b, t: (b * T + t, 0, 0)),
            pl.BlockSpec((1, 1, TILE_R), lambda b, t: (b * T + t, 0, 0)),
            pl.BlockSpec((1, TILE_R, H, W), lambda b, t: (b, t, 0, 0)),
            pl.BlockSpec((1, 1, H, W), lambda b, t: (b, 0, 0, 0)),
        ],
        out_specs=pl.BlockSpec((1, 1), lambda b, t: (0, 0)),
        out_shape=jax.ShapeDtypeStruct((1, 1), jnp.float32),
        scratch_shapes=[
            pltpu.SMEM((1,), jnp.float32),
            pltpu.SMEM((1,), jnp.float32),
        ],
    )(row, col, response_map, boundaries)
    return out[0, 0]


# speedup vs baseline: 1.5447x; 1.0117x over previous
import functools

import jax
import jax.numpy as jnp
from jax import lax
from jax.experimental import pallas as pl
from jax.experimental.pallas import tpu as pltpu

EPS_ = 1e-10


def _loss_kernel(row_ref, col_ref, rm_ref, b_ref, out_ref, num_acc, den_acc,
                 *, tile_r, h, w, nb, nt):
    b = pl.program_id(0)
    t = pl.program_id(1)

    @pl.when(jnp.logical_and(b == 0, t == 0))
    def _init():
        num_acc[0] = 0.0
        den_acc[0] = 0.0

    x = rm_ref[0]
    bmap = b_ref[0, 0]
    row = row_ref[0, 0]
    col = col_ref[0, 0]

    iota_w = lax.broadcasted_iota(jnp.int32, (tile_r, 1, w), 2)
    mask_w = iota_w == col[:, None, None]
    iota_h = lax.broadcasted_iota(jnp.int32, (tile_r, h), 1)
    mask_h = iota_h == row[:, None]

    sum_w = jnp.sum(x, axis=2)
    denom = jnp.sum(sum_w, axis=1)

    srm_w = jnp.sum(jnp.where(mask_w, x, 0.0), axis=2)
    srm = jnp.sum(jnp.where(mask_h, srm_w, 0.0), axis=1)

    sb_w = jnp.sum(jnp.where(mask_w, bmap[None], 0.0), axis=2)
    sb = jnp.sum(jnp.where(mask_h, sb_w, 0.0), axis=1)

    num_acc[0] += jnp.sum(sb * -jnp.log(EPS_ + srm / denom))
    den_acc[0] += jnp.sum(sb)

    @pl.when(jnp.logical_and(b == nb - 1, t == nt - 1))
    def _fin():
        out_ref[...] = jnp.full((1, 1), num_acc[0] / (1.0 + den_acc[0]), jnp.float32)


def kernel(response_map, source_feature_1d_locations, boundaries):
    B, S, H, W = response_map.shape
    TILE_R = 128
    T = S // TILE_R

    loc = source_feature_1d_locations.astype(jnp.int32)
    row = (loc // W).reshape(B * T, 1, TILE_R)
    col = (loc % W).reshape(B * T, 1, TILE_R)

    out = pl.pallas_call(
        functools.partial(_loss_kernel, tile_r=TILE_R, h=H, w=W, nb=B, nt=T),
        grid=(B, T),
        in_specs=[
            pl.BlockSpec((1, 1, TILE_R), lambda b, t: (b * T + t, 0, 0)),
            pl.BlockSpec((1, 1, TILE_R), lambda b, t: (b * T + t, 0, 0)),
            pl.BlockSpec((1, TILE_R, H, W), lambda b, t: (b, t, 0, 0)),
            pl.BlockSpec((1, 1, H, W), lambda b, t: (b, 0, 0, 0)),
        ],
        out_specs=pl.BlockSpec((1, 1), lambda b, t: (0, 0)),
        out_shape=jax.ShapeDtypeStruct((1, 1), jnp.float32),
        scratch_shapes=[
            pltpu.SMEM((1,), jnp.float32),
            pltpu.SMEM((1,), jnp.float32),
        ],
    )(row, col, response_map, boundaries)
    return out[0, 0]
